# baseline (speedup 1.0000x reference)
import functools
import jax
import jax.numpy as jnp
from jax import lax
from jax.experimental import pallas as pl
from jax.experimental.pallas import tpu as pltpu
from jax.experimental.pallas import tpu_sc as plsc

_NW = 32
_NCHUNK = 8


def kernel(sample, epoch, epoch_table):
    batch, nfeat = sample.shape
    nvocab, nemb = epoch_table.shape
    nout = nfeat + nemb
    nrows = batch // _NW
    chunk = nrows // _NCHUNK
    nj = nemb // 16

    epoch_idx = jnp.full((16,), jnp.asarray(epoch, jnp.int32))

    mesh = plsc.VectorSubcoreMesh(core_axis_name="c", subcore_axis_name="s")

    @functools.partial(
        pl.kernel, mesh=mesh,
        out_type=jax.ShapeDtypeStruct((batch, nout), jnp.float32),
        scratch_types=[
            pltpu.VMEM((16,), jnp.int32),
            pltpu.VMEM((nvocab, nemb), jnp.float32),
            pltpu.VMEM((chunk, nemb), jnp.float32),
            pltpu.VMEM((_NCHUNK, chunk, nfeat), jnp.float32),
            pltpu.SemaphoreType.DMA((_NCHUNK,)),
            pltpu.SemaphoreType.DMA((_NCHUNK,)),
            pltpu.SemaphoreType.DMA((_NCHUNK,)),
        ],
    )
    def k(epoch_hbm, table_hbm, sample_hbm, out_hbm,
          idx_v, table_v, bcast_v, samp_v, sem_in, sem_out, sem_bc):
        wid = lax.axis_index("s") * 2 + lax.axis_index("c")
        base = wid * nrows

        for c in range(_NCHUNK):
            pltpu.make_async_copy(
                sample_hbm.at[pl.ds(base + c * chunk, chunk), :],
                samp_v.at[c],
                sem_in.at[c],
            ).start()

        pltpu.sync_copy(table_hbm, table_v)
        pltpu.sync_copy(epoch_hbm, idx_v)
        e_scalar = idx_v[...][0]
        row_regs = [table_v[e_scalar, pl.ds(16 * j, 16)] for j in range(nj)]

        def fill_row(r, carry):
            for j in range(nj):
                bcast_v[r, pl.ds(16 * j, 16)] = row_regs[j]
            return carry

        lax.fori_loop(0, chunk, fill_row, 0)

        for c in range(_NCHUNK):
            pltpu.make_async_copy(
                bcast_v,
                out_hbm.at[pl.ds(base + c * chunk, chunk), pl.ds(nfeat, nemb)],
                sem_bc.at[c],
            ).start()

        for c in range(_NCHUNK):
            pltpu.make_async_copy(
                sample_hbm.at[pl.ds(base + c * chunk, chunk), :],
                samp_v.at[c],
                sem_in.at[c],
            ).wait()
            pltpu.make_async_copy(
                samp_v.at[c],
                out_hbm.at[pl.ds(base + c * chunk, chunk), pl.ds(0, nfeat)],
                sem_out.at[c],
            ).start()
        for c in range(_NCHUNK):
            pltpu.make_async_copy(
                samp_v.at[c],
                out_hbm.at[pl.ds(base + c * chunk, chunk), pl.ds(0, nfeat)],
                sem_out.at[c],
            ).wait()
            pltpu.make_async_copy(
                bcast_v,
                out_hbm.at[pl.ds(base + c * chunk, chunk), pl.ds(nfeat, nemb)],
                sem_bc.at[c],
            ).wait()

    return k(epoch_idx, epoch_table, sample)


# speedup vs baseline: 1.9883x; 1.9883x over previous
import jax
import jax.numpy as jnp
from jax.experimental import pallas as pl
from jax.experimental.pallas import tpu as pltpu

_BLOCK = 8192


def _concat_kernel(epoch_ref, table_ref, sample_ref, out_ref):
    e = epoch_ref[0]
    row = table_ref[pl.ds(e, 1), :]
    nf = sample_ref.shape[1]
    out_ref[:, :nf] = sample_ref[...]
    out_ref[:, nf:] = jnp.broadcast_to(row, (out_ref.shape[0], row.shape[1]))


def kernel(sample, epoch, epoch_table):
    batch, nfeat = sample.shape
    nvocab, nemb = epoch_table.shape
    epoch_arr = jnp.asarray(epoch, jnp.int32).reshape((1,))
    nout = nfeat + nemb
    block = _BLOCK
    while batch % block:
        block //= 2
    grid = (batch // block,)
    return pl.pallas_call(
        _concat_kernel,
        grid=grid,
        in_specs=[
            pl.BlockSpec(memory_space=pltpu.SMEM),
            pl.BlockSpec((nvocab, nemb), lambda i: (0, 0)),
            pl.BlockSpec((block, nfeat), lambda i: (i, 0)),
        ],
        out_specs=pl.BlockSpec((block, nout), lambda i: (i, 0)),
        out_shape=jax.ShapeDtypeStruct((batch, nout), sample.dtype),
        compiler_params=pltpu.CompilerParams(
            dimension_semantics=("parallel",),
        ),
    )(epoch_arr, epoch_table, sample)
